# baseline (speedup 1.0000x reference)
import jax
import jax.numpy as jnp
import numpy as np
from jax import lax
from jax.experimental import pallas as pl
from jax.experimental.pallas import tpu as pltpu
from jax.experimental.pallas import tpu_sc as plsc

_D = 64
_S = 200
_B = 4096
_NC = 2
_NS = 16
_NW = _NC * _NS
_BPW = _B // _NW
_NBUF = 4
_OBUF = 2
_LEAD = 2


def _positional_encoding_bcast(n=10000.0):
    position = np.arange(_S, dtype=np.float32)[:, None]
    division_term = np.exp(
        np.arange(0, _D, 2, dtype=np.float32) * (-np.log(n) / _D)
    )
    pos_enc = np.zeros((_S, _D), dtype=np.float32)
    pos_enc[:, 0::2] = np.sin(position * division_term)
    pos_enc[:, 1::2] = np.cos(position * division_term)
    bcast = np.repeat(pos_enc, 16, axis=1).reshape(_S, 8, 128)
    return jnp.asarray(bcast)


def _sc_body(xt_hbm, tab_hbm, pos_hbm, out_hbm, xi_v, ri_v, ho_v, rows_v,
             ob_v, pos_v, gsem, ssem):
    wid = lax.axis_index("s") * _NC + lax.axis_index("c")
    bw = wid * _BPW

    def issue_gather(b, s):
        o = b * _BPW
        pltpu.sync_copy(xt_hbm.at[pl.ds(s * _B + bw, _BPW)],
                        xi_v.at[pl.ds(o, _BPW)])
        for g in range(_BPW // 16):
            sl = pl.ds(o + g * 16, 16)
            v = xi_v[sl]
            ri_v[sl] = lax.shift_right_logical(v, 1)
            ho_v[sl] = lax.shift_left((v & 1), 6)
        pltpu.async_copy(
            tab_hbm.at[ri_v.at[pl.ds(o, _BPW)]],
            rows_v.at[b],
            gsem.at[b],
        )
        pltpu.async_copy(pos_hbm.at[s], pos_v.at[b], gsem.at[b])

    def wait_gather(b):
        o = b * _BPW
        pltpu.make_async_copy(
            tab_hbm.at[ri_v.at[pl.ds(o, _BPW)]],
            rows_v.at[b],
            gsem.at[b],
        ).wait()
        pltpu.make_async_copy(pos_hbm.at[0], pos_v.at[b],
                              gsem.at[b]).wait()

    def start_store(ob, s):
        pltpu.async_copy(ob_v.at[ob], out_hbm.at[s, :, pl.ds(bw, _BPW)],
                         ssem.at[ob])

    def wait_store(ob, s):
        pltpu.make_async_copy(ob_v.at[ob],
                              out_hbm.at[s, :, pl.ds(bw, _BPW)],
                              ssem.at[ob]).wait()

    for b in range(_LEAD):
        issue_gather(b, b)

    iota16 = lax.iota(jnp.int32, 16)

    def step(k, b, ob):
        wait_gather(b)

        @pl.when(k >= _OBUF)
        def _():
            wait_store(ob, k - _OBUF)

        o = b * _BPW
        hoffs = [ho_v[pl.ds(o + g * 16, 16)] for g in range(_BPW // 16)]
        trows = [iota16 + g * 16 for g in range(_BPW // 16)]

        def dloop(d, carry):
            pv = pos_v[b, lax.shift_right_logical(d, 3),
                       pl.ds(lax.shift_left(d & 7, 4), 16)]
            for g in range(_BPW // 16):
                vals = plsc.load_gather(rows_v.at[b], [trows[g], hoffs[g] + d])
                ob_v[ob, d, pl.ds(g * 16, 16)] = vals + pv
            return carry

        lax.fori_loop(0, _D, dloop, 0, unroll=2)
        start_store(ob, k)

        nxt = k + _LEAD

        @pl.when(nxt < _S)
        def _():
            issue_gather(nxt % _NBUF, nxt)

    def outer(io, carry):
        for u in range(_NBUF):
            k = io * _NBUF + u
            step(k, u, k % _OBUF)
        return carry

    lax.fori_loop(0, _S // _NBUF, outer, 0)
    wait_store(0, _S - 2)
    wait_store(1, _S - 1)


def kernel(x, embedding_table):
    B, S = x.shape
    xt = jnp.transpose(x).reshape(B * S).astype(jnp.int32)
    tab2 = embedding_table.reshape(embedding_table.shape[0] // 2, 2 * _D)
    posp = _positional_encoding_bcast()

    mesh = plsc.VectorSubcoreMesh(core_axis_name="c", subcore_axis_name="s")
    out_t = pl.kernel(
        _sc_body,
        out_type=jax.ShapeDtypeStruct((S, _D, B), jnp.float32),
        mesh=mesh,
        scratch_types=[
            pltpu.VMEM((_NBUF * _BPW,), jnp.int32),
            pltpu.VMEM((_NBUF * _BPW,), jnp.int32),
            pltpu.VMEM((_NBUF * _BPW,), jnp.int32),
            pltpu.VMEM((_NBUF, _BPW, 2 * _D), jnp.float32),
            pltpu.VMEM((_OBUF, _D, _BPW), jnp.float32),
            pltpu.VMEM((_NBUF, 8, 128), jnp.float32),
            pltpu.SemaphoreType.DMA((_NBUF,)),
            pltpu.SemaphoreType.DMA((_OBUF,)),
        ],
        compiler_params=pltpu.CompilerParams(use_tc_tiling_on_sc=True,
                                             needs_layout_passes=False),
    )(xt, tab2, posp)
    return jnp.transpose(out_t, (2, 0, 1))


# speedup vs baseline: 1.4705x; 1.4705x over previous
import jax
import jax.numpy as jnp
import numpy as np
from jax import lax
from jax.experimental import pallas as pl
from jax.experimental.pallas import tpu as pltpu
from jax.experimental.pallas import tpu_sc as plsc

_D = 64
_S = 200
_B = 4096
_NC = 2
_NS = 16
_NW = _NC * _NS
_BPW = _B // _NW
_NBUF = 4
_OBUF = 2
_LEAD = 2


def _positional_encoding_flat(n=10000.0):
    position = np.arange(_S, dtype=np.float32)[:, None]
    division_term = np.exp(
        np.arange(0, _D, 2, dtype=np.float32) * (-np.log(n) / _D)
    )
    pos_enc = np.zeros((_S, _D), dtype=np.float32)
    pos_enc[:, 0::2] = np.sin(position * division_term)
    pos_enc[:, 1::2] = np.cos(position * division_term)
    return jnp.asarray(pos_enc.reshape(_S * _D))


def _sc_body(xt_hbm, tab_hbm, pos_hbm, out_hbm, xi_v, ri_v, ho_v, rows_v,
             ob_v, pos_v, gsem, ssem):
    wid = lax.axis_index("s") * _NC + lax.axis_index("c")
    bw = wid * _BPW

    def issue_gather(b, s):
        o = b * _BPW
        pltpu.sync_copy(xt_hbm.at[pl.ds(s * _B + bw, _BPW)],
                        xi_v.at[pl.ds(o, _BPW)])
        for g in range(_BPW // 16):
            sl = pl.ds(o + g * 16, 16)
            v = xi_v[sl]
            ri_v[sl] = lax.shift_right_logical(v, 1)
            ho_v[sl] = lax.shift_left((v & 1), 6)
        pltpu.async_copy(
            tab_hbm.at[ri_v.at[pl.ds(o, _BPW)]],
            rows_v.at[b],
            gsem.at[b],
        )
        pltpu.async_copy(pos_hbm.at[pl.ds(s * _D, _D)],
                         pos_v.at[pl.ds(b * _D, _D)], gsem.at[b])

    def wait_gather(b):
        o = b * _BPW
        pltpu.make_async_copy(
            tab_hbm.at[ri_v.at[pl.ds(o, _BPW)]],
            rows_v.at[b],
            gsem.at[b],
        ).wait()
        pltpu.make_async_copy(pos_hbm.at[pl.ds(0, _D)],
                              pos_v.at[pl.ds(b * _D, _D)],
                              gsem.at[b]).wait()

    def start_store(ob, s):
        pltpu.async_copy(ob_v.at[ob], out_hbm.at[s, :, pl.ds(bw, _BPW)],
                         ssem.at[ob])

    def wait_store(ob, s):
        pltpu.make_async_copy(ob_v.at[ob],
                              out_hbm.at[s, :, pl.ds(bw, _BPW)],
                              ssem.at[ob]).wait()

    for b in range(_LEAD):
        issue_gather(b, b)

    iota16 = lax.iota(jnp.int32, 16)

    def step(k, b, ob):
        wait_gather(b)

        @pl.when(k >= _OBUF)
        def _():
            wait_store(ob, k - _OBUF)

        o = b * _BPW
        rows2 = rows_v.at[b]
        ob2 = ob_v.at[ob]

        def tblock(g, carry):
            t0 = g * 16
            trow = iota16 + t0
            hos = ho_v[pl.ds(o + t0, 16)]
            for d0 in range(0, _D, 16):
                for kd in range(16):
                    diag = (iota16 + kd) & 15
                    drow = diag + d0
                    vals = plsc.load_gather(rows2, [trow, hos + drow])
                    pv = plsc.load_gather(pos_v, [drow + b * _D])
                    plsc.store_scatter(ob2, [drow, trow], vals + pv)
            return carry

        lax.fori_loop(0, _BPW // 16, tblock, 0)
        start_store(ob, k)

        nxt = k + _LEAD

        @pl.when(nxt < _S)
        def _():
            issue_gather(nxt % _NBUF, nxt)

    def outer(io, carry):
        for u in range(_NBUF):
            k = io * _NBUF + u
            step(k, u, k % _OBUF)
        return carry

    lax.fori_loop(0, _S // _NBUF, outer, 0)
    wait_store(0, _S - 2)
    wait_store(1, _S - 1)


def kernel(x, embedding_table):
    B, S = x.shape
    xt = jnp.transpose(x).reshape(B * S).astype(jnp.int32)
    tab2 = embedding_table.reshape(embedding_table.shape[0] // 2, 2 * _D)
    posp = _positional_encoding_flat()

    mesh = plsc.VectorSubcoreMesh(core_axis_name="c", subcore_axis_name="s")
    out_t = pl.kernel(
        _sc_body,
        out_type=jax.ShapeDtypeStruct((S, _D, B), jnp.float32),
        mesh=mesh,
        scratch_types=[
            pltpu.VMEM((_NBUF * _BPW,), jnp.int32),
            pltpu.VMEM((_NBUF * _BPW,), jnp.int32),
            pltpu.VMEM((_NBUF * _BPW,), jnp.int32),
            pltpu.VMEM((_NBUF, _BPW, 2 * _D), jnp.float32),
            pltpu.VMEM((_OBUF, _D, _BPW), jnp.float32),
            pltpu.VMEM((_NBUF * _D,), jnp.float32),
            pltpu.SemaphoreType.DMA((_NBUF,)),
            pltpu.SemaphoreType.DMA((_OBUF,)),
        ],
        compiler_params=pltpu.CompilerParams(use_tc_tiling_on_sc=True,
                                             needs_layout_passes=False),
    )(xt, tab2, posp)
    return jnp.transpose(out_t, (2, 0, 1))


# speedup vs baseline: 1.7894x; 1.2169x over previous
import jax
import jax.numpy as jnp
import numpy as np
from jax import lax
from jax.experimental import pallas as pl
from jax.experimental.pallas import tpu as pltpu
from jax.experimental.pallas import tpu_sc as plsc

_D = 64
_S = 200
_B = 4096
_NC = 2
_NS = 16
_NW = _NC * _NS
_BPW = _B // _NW
_NBUF = 4
_OBUF = 2
_LEAD = 2


def _positional_encoding_flat(n=10000.0):
    position = np.arange(_S, dtype=np.float32)[:, None]
    division_term = np.exp(
        np.arange(0, _D, 2, dtype=np.float32) * (-np.log(n) / _D)
    )
    pos_enc = np.zeros((_S, _D), dtype=np.float32)
    pos_enc[:, 0::2] = np.sin(position * division_term)
    pos_enc[:, 1::2] = np.cos(position * division_term)
    return jnp.asarray(pos_enc.reshape(_S * _D))


def _sc_body(xt_hbm, tab_hbm, pos_hbm, out_hbm, xi_v, ri_v, ho_v, rows_v,
             ob_v, pos_v, gsem, ssem, isem):
    wid = lax.axis_index("s") * _NC + lax.axis_index("c")
    bw = wid * _BPW

    def issue_idx(b, s):
        o = b * _BPW
        pltpu.async_copy(xt_hbm.at[pl.ds(s * _B + bw, _BPW)],
                         xi_v.at[pl.ds(o, _BPW)], isem.at[b])

    def issue_gather(b, s):
        o = b * _BPW
        pltpu.make_async_copy(xt_hbm.at[pl.ds(s * _B + bw, _BPW)],
                              xi_v.at[pl.ds(o, _BPW)], isem.at[b]).wait()
        for g in range(_BPW // 16):
            sl = pl.ds(o + g * 16, 16)
            v = xi_v[sl]
            ri_v[sl] = lax.shift_right_logical(v, 1)
            ho_v[sl] = lax.shift_left((v & 1), 6)
        pltpu.async_copy(
            tab_hbm.at[ri_v.at[pl.ds(o, _BPW)]],
            rows_v.at[b],
            gsem.at[b],
        )
        pltpu.async_copy(pos_hbm.at[pl.ds(s * _D, _D)],
                         pos_v.at[pl.ds(b * _D, _D)], gsem.at[b])

    def wait_gather(b):
        o = b * _BPW
        pltpu.make_async_copy(
            tab_hbm.at[ri_v.at[pl.ds(o, _BPW)]],
            rows_v.at[b],
            gsem.at[b],
        ).wait()
        pltpu.make_async_copy(pos_hbm.at[pl.ds(0, _D)],
                              pos_v.at[pl.ds(b * _D, _D)],
                              gsem.at[b]).wait()

    def start_store(ob, s):
        pltpu.async_copy(ob_v.at[ob], out_hbm.at[s, :, pl.ds(bw, _BPW)],
                         ssem.at[ob])

    def wait_store(ob, s):
        pltpu.make_async_copy(ob_v.at[ob],
                              out_hbm.at[s, :, pl.ds(bw, _BPW)],
                              ssem.at[ob]).wait()

    for b in range(_LEAD + 1):
        issue_idx(b, b)
    for b in range(_LEAD):
        issue_gather(b, b)

    iota16 = lax.iota(jnp.int32, 16)

    def step(k, b, ob):
        wait_gather(b)

        @pl.when(k >= _OBUF)
        def _():
            wait_store(ob, k - _OBUF)

        o = b * _BPW
        rows2 = rows_v.at[b]
        ob2 = ob_v.at[ob]
        hos_l = [ho_v[pl.ds(o + t * 16, 16)] for t in range(_BPW // 16)]
        trow_l = [iota16 + t * 16 for t in range(_BPW // 16)]

        def dblock(i, carry):
            kd = lax.shift_right_logical(i, 2)
            d0 = lax.shift_left(i & 3, 4)
            drow = ((iota16 + kd) & 15) + d0
            pv = plsc.load_gather(pos_v, [drow + b * _D])
            for t in range(_BPW // 16):
                vals = plsc.load_gather(rows2, [trow_l[t], hos_l[t] + drow])
                plsc.store_scatter(ob2, [drow, trow_l[t]], vals + pv)
            return carry

        lax.fori_loop(0, 4 * 16, dblock, 0, unroll=2)
        start_store(ob, k)

        nxt = k + _LEAD

        @pl.when(nxt + 1 < _S)
        def _():
            issue_idx((nxt + 1) % _NBUF, nxt + 1)

        @pl.when(nxt < _S)
        def _():
            issue_gather(nxt % _NBUF, nxt)

    def outer(io, carry):
        for u in range(_NBUF):
            k = io * _NBUF + u
            step(k, u, k % _OBUF)
        return carry

    lax.fori_loop(0, _S // _NBUF, outer, 0)
    wait_store(0, _S - 2)
    wait_store(1, _S - 1)


def kernel(x, embedding_table):
    B, S = x.shape
    xt = jnp.transpose(x).reshape(B * S).astype(jnp.int32)
    tab2 = embedding_table.reshape(embedding_table.shape[0] // 2, 2 * _D)
    posp = _positional_encoding_flat()

    mesh = plsc.VectorSubcoreMesh(core_axis_name="c", subcore_axis_name="s")
    out_t = pl.kernel(
        _sc_body,
        out_type=jax.ShapeDtypeStruct((S, _D, B), jnp.float32),
        mesh=mesh,
        scratch_types=[
            pltpu.VMEM((_NBUF * _BPW,), jnp.int32),
            pltpu.VMEM((_NBUF * _BPW,), jnp.int32),
            pltpu.VMEM((_NBUF * _BPW,), jnp.int32),
            pltpu.VMEM((_NBUF, _BPW, 2 * _D), jnp.float32),
            pltpu.VMEM((_OBUF, _D, _BPW), jnp.float32),
            pltpu.VMEM((_NBUF * _D,), jnp.float32),
            pltpu.SemaphoreType.DMA((_NBUF,)),
            pltpu.SemaphoreType.DMA((_OBUF,)),
            pltpu.SemaphoreType.DMA((_NBUF,)),
        ],
        compiler_params=pltpu.CompilerParams(use_tc_tiling_on_sc=True,
                                             needs_layout_passes=False,
                                             disable_bounds_checks=True),
    )(xt, tab2, posp)
    return jnp.transpose(out_t, (2, 0, 1))


# speedup vs baseline: 1.9393x; 1.0837x over previous
import jax
import jax.numpy as jnp
import numpy as np
from jax import lax
from jax.experimental import pallas as pl
from jax.experimental.pallas import tpu as pltpu
from jax.experimental.pallas import tpu_sc as plsc

_D = 64
_S = 200
_B = 4096
_NC = 2
_NS = 16
_NW = _NC * _NS
_BPW = _B // _NW
_NBUF = 4
_OBUF = 2
_LEAD = 3


def _positional_encoding_flat(n=10000.0):
    position = np.arange(_S, dtype=np.float32)[:, None]
    division_term = np.exp(
        np.arange(0, _D, 2, dtype=np.float32) * (-np.log(n) / _D)
    )
    pos_enc = np.zeros((_S, _D), dtype=np.float32)
    pos_enc[:, 0::2] = np.sin(position * division_term)
    pos_enc[:, 1::2] = np.cos(position * division_term)
    return jnp.asarray(pos_enc.reshape(_S * _D))


_TCH = 7812


def _pack_body(tt_hbm, tail_hbm, tp_hbm, blk_v, pk_v, rsem, wsem):
    wid = lax.axis_index("s") * _NC + lax.axis_index("c")
    iota16 = lax.iota(jnp.int32, 16)

    @pl.when(wid == 0)
    def _():
        pltpu.sync_copy(tail_hbm, pk_v.at[0, pl.ds(0, 32)])
        pltpu.sync_copy(pk_v.at[0, pl.ds(0, 32)],
                        tp_hbm.at[pl.ds(_TCH * 64, 32)])

    def issue_read(b, c):
        pltpu.async_copy(tt_hbm.at[:, pl.ds(c * 128, 128)], blk_v.at[b],
                         rsem.at[b])

    def wait_read(b, c):
        pltpu.make_async_copy(tt_hbm.at[:, pl.ds(c * 128, 128)],
                              blk_v.at[b], rsem.at[b]).wait()

    def chunk_at(i):
        return wid + _NW * i

    for b in range(2):
        @pl.when(chunk_at(b) < _TCH)
        def _():
            issue_read(b, chunk_at(b))

    niter = (_TCH + _NW - 1) // _NW

    def loop(i, carry):
        c = chunk_at(i)

        @pl.when(c < _TCH)
        def _():
            b = i % 2
            pb = i % 2
            wait_read(b, c)

            @pl.when(i >= 2)
            def _():
                pltpu.make_async_copy(
                    pk_v.at[pb + 1],
                    tp_hbm.at[pl.ds((c - 2 * _NW) * 64, 64)],
                    wsem.at[pb]).wait()

            blk2 = blk_v.at[b]
            pk2 = pk_v.at[pb + 1]

            def dblk(j, c2):
                rr0 = lax.shift_left(lax.shift_right_logical(j, 3), 4)
                c20 = (j & 7)
                hc = lax.shift_right_logical(c20, 2)
                srow = ((lax.shift_left(c20, 4) + iota16) & 63)
                for kd in range(16):
                    orow = ((iota16 + kd) & 15) + rr0
                    scol = lax.shift_left(orow, 1) + hc
                    vals = plsc.load_gather(blk2, [srow, scol])
                    plsc.store_scatter(pk2, [orow,
                                             lax.shift_left(c20, 4) + iota16],
                                       vals)
                return c2

            lax.fori_loop(0, 32, dblk, 0)
            pltpu.async_copy(pk2, tp_hbm.at[pl.ds(c * 64, 64)], wsem.at[pb])

            @pl.when(chunk_at(i + 2) < _TCH)
            def _():
                issue_read(b, chunk_at(i + 2))

        return carry

    lax.fori_loop(0, niter, loop, 0)
    def drain(pb, c):
        @pl.when(c < _TCH)
        def _():
            pltpu.make_async_copy(pk_v.at[pb + 1],
                                  tp_hbm.at[pl.ds(c * 64, 64)],
                                  wsem.at[pb]).wait()

    drain((niter - 2) % 2, chunk_at(niter - 2))
    drain((niter - 1) % 2, chunk_at(niter - 1))


def _sc_body(xt_hbm, tab_hbm, pos_hbm, out_hbm, xi_v, ri_v, ho_v, rows_v,
             ob_v, pos_v, gsem, ssem, isem):
    wid = lax.axis_index("s") * _NC + lax.axis_index("c")
    bw = wid * _BPW

    def issue_idx(b, s):
        o = b * _BPW
        pltpu.async_copy(xt_hbm.at[pl.ds(s * _B + bw, _BPW)],
                         xi_v.at[pl.ds(o, _BPW)], isem.at[b])

    def issue_gather(b, s):
        o = b * _BPW
        pltpu.make_async_copy(xt_hbm.at[pl.ds(s * _B + bw, _BPW)],
                              xi_v.at[pl.ds(o, _BPW)], isem.at[b]).wait()
        for g in range(_BPW // 16):
            sl = pl.ds(o + g * 16, 16)
            v = xi_v[sl]
            ri_v[sl] = lax.shift_right_logical(v, 1)
            ho_v[sl] = lax.shift_left((v & 1), 6)
        pltpu.async_copy(
            tab_hbm.at[ri_v.at[pl.ds(o, _BPW)]],
            rows_v.at[b],
            gsem.at[b],
        )
        pltpu.async_copy(pos_hbm.at[pl.ds(s * _D, _D)],
                         pos_v.at[pl.ds(b * _D, _D)], gsem.at[b])

    def wait_gather(b):
        o = b * _BPW
        pltpu.make_async_copy(
            tab_hbm.at[ri_v.at[pl.ds(o, _BPW)]],
            rows_v.at[b],
            gsem.at[b],
        ).wait()
        pltpu.make_async_copy(pos_hbm.at[pl.ds(0, _D)],
                              pos_v.at[pl.ds(b * _D, _D)],
                              gsem.at[b]).wait()

    def start_store(ob, s):
        pltpu.async_copy(ob_v.at[ob], out_hbm.at[s, :, pl.ds(bw, _BPW)],
                         ssem.at[ob])

    def wait_store(ob, s):
        pltpu.make_async_copy(ob_v.at[ob],
                              out_hbm.at[s, :, pl.ds(bw, _BPW)],
                              ssem.at[ob]).wait()

    for b in range(_LEAD + 1):
        issue_idx(b, b)
    for b in range(_LEAD):
        issue_gather(b, b)

    iota16 = lax.iota(jnp.int32, 16)

    def step(k, b, ob):
        wait_gather(b)

        @pl.when(k >= _OBUF)
        def _():
            wait_store(ob, k - _OBUF)

        o = b * _BPW
        rows2 = rows_v.at[b]
        ob2 = ob_v.at[ob]
        hos_l = [ho_v[pl.ds(o + t * 16, 16)] for t in range(_BPW // 16)]
        trow_l = [iota16 + t * 16 for t in range(_BPW // 16)]

        def dblock(i, carry):
            kd = lax.shift_right_logical(i, 2)
            d0 = lax.shift_left(i & 3, 4)
            drow = ((iota16 + kd) & 15) + d0
            pv = plsc.load_gather(pos_v, [drow + b * _D])
            for t in range(_BPW // 16):
                vals = plsc.load_gather(rows2, [trow_l[t], hos_l[t] + drow])
                plsc.store_scatter(ob2, [drow, trow_l[t]], vals + pv)
            return carry

        lax.fori_loop(0, 4 * 16, dblock, 0, unroll=2)
        start_store(ob, k)

        nxt = k + _LEAD

        @pl.when(nxt + 1 < _S)
        def _():
            issue_idx((nxt + 1) % _NBUF, nxt + 1)

        @pl.when(nxt < _S)
        def _():
            issue_gather(nxt % _NBUF, nxt)

    def outer(io, carry):
        for u in range(_NBUF):
            k = io * _NBUF + u
            step(k, u, k % _OBUF)
        return carry

    lax.fori_loop(0, _S // _NBUF, outer, 0)
    wait_store(0, _S - 2)
    wait_store(1, _S - 1)


def kernel(x, embedding_table):
    B, S = x.shape
    V = embedding_table.shape[0]
    xt = jnp.transpose(x).reshape(B * S).astype(jnp.int32)
    posp = _positional_encoding_flat()

    mesh = plsc.VectorSubcoreMesh(core_axis_name="c", subcore_axis_name="s")
    tabT = jnp.transpose(embedding_table)
    tail = embedding_table[_TCH * 128:].reshape(32, 2 * _D)
    tab2 = pl.kernel(
        _pack_body,
        out_type=jax.ShapeDtypeStruct((V // 2, 2 * _D), jnp.float32),
        mesh=mesh,
        scratch_types=[
            pltpu.VMEM((2, _D, 128), jnp.float32),
            pltpu.VMEM((3, _D, 128), jnp.float32),
            pltpu.SemaphoreType.DMA((2,)),
            pltpu.SemaphoreType.DMA((2,)),
        ],
        compiler_params=pltpu.CompilerParams(use_tc_tiling_on_sc=True,
                                             needs_layout_passes=False,
                                             disable_bounds_checks=True),
    )(tabT, tail)
    out_t = pl.kernel(
        _sc_body,
        out_type=jax.ShapeDtypeStruct((S, _D, B), jnp.float32),
        mesh=mesh,
        scratch_types=[
            pltpu.VMEM((_NBUF * _BPW,), jnp.int32),
            pltpu.VMEM((_NBUF * _BPW,), jnp.int32),
            pltpu.VMEM((_NBUF * _BPW,), jnp.int32),
            pltpu.VMEM((_NBUF, _BPW, 2 * _D), jnp.float32),
            pltpu.VMEM((_OBUF, _D, _BPW), jnp.float32),
            pltpu.VMEM((_NBUF * _D,), jnp.float32),
            pltpu.SemaphoreType.DMA((_NBUF,)),
            pltpu.SemaphoreType.DMA((_OBUF,)),
            pltpu.SemaphoreType.DMA((_NBUF,)),
        ],
        compiler_params=pltpu.CompilerParams(use_tc_tiling_on_sc=True,
                                             needs_layout_passes=False,
                                             disable_bounds_checks=True),
    )(xt, tab2, posp)
    return jnp.transpose(out_t, (2, 0, 1))
